# baseline (speedup 1.0000x reference)
import jax
import jax.numpy as jnp
from jax.experimental import pallas as pl
from jax.experimental.pallas import tpu as pltpu

TS_START = 26
N_TABLE = 126
N_REP = 20
N_TS = N_TABLE - TS_START
F_OUT = TS_START + N_TS * N_REP
DIM = 64
W = F_OUT * DIM
B_BLK = 8


def _bias_kernel(table_ref, full_ref):
    full_ref[0:TS_START] = table_ref[0:TS_START]
    ts = table_ref[TS_START:N_TABLE]
    for r in range(N_REP):
        base = TS_START + r * N_TS
        full_ref[base:base + N_TS] = ts


def _add_kernel(x_ref, b_ref, o_ref):
    o_ref[...] = x_ref[...] + b_ref[...]


def kernel(X, table):
    B = X.shape[0]
    full2d = pl.pallas_call(
        _bias_kernel,
        out_shape=jax.ShapeDtypeStruct((F_OUT, DIM), table.dtype),
    )(table)
    bias_row = full2d.reshape(1, W)
    X2 = X.reshape(B, W)
    out = pl.pallas_call(
        _add_kernel,
        grid=(B // B_BLK,),
        in_specs=[
            pl.BlockSpec((B_BLK, W), lambda i: (i, 0)),
            pl.BlockSpec((1, W), lambda i: (0, 0)),
        ],
        out_specs=pl.BlockSpec((B_BLK, W), lambda i: (i, 0)),
        out_shape=jax.ShapeDtypeStruct((B, W), X.dtype),
        compiler_params=pltpu.CompilerParams(
            dimension_semantics=("parallel",),
        ),
    )(X2, bias_row)
    return out.reshape(B, F_OUT, DIM)


# speedup vs baseline: 1.0014x; 1.0014x over previous
import jax
import jax.numpy as jnp
from jax import lax
from jax.experimental import pallas as pl
from jax.experimental.pallas import tpu as pltpu

TS_START = 26
N_TABLE = 126
N_REP = 20
N_TS = N_TABLE - TS_START
F_OUT = TS_START + N_TS * N_REP
DIM = 64
W = F_OUT * DIM
B_BLK = 8
DEPTH = 4


def _bias_kernel(table_ref, full_ref):
    full_ref[0:TS_START] = table_ref[0:TS_START]
    ts = table_ref[TS_START:N_TABLE]
    for r in range(N_REP):
        base = TS_START + r * N_TS
        full_ref[base:base + N_TS] = ts


def _stream_kernel(x_hbm, bias_ref, o_hbm, in_buf, out_buf, in_sems, out_sems):
    n_blocks = x_hbm.shape[0] // B_BLK

    def in_copy(i, slot):
        return pltpu.make_async_copy(
            x_hbm.at[pl.ds(i * B_BLK, B_BLK)], in_buf.at[slot], in_sems.at[slot])

    def out_copy(i, slot):
        return pltpu.make_async_copy(
            out_buf.at[slot], o_hbm.at[pl.ds(i * B_BLK, B_BLK)], out_sems.at[slot])

    for d in range(DEPTH):
        in_copy(d, d).start()

    def step(i, carry):
        slot = lax.rem(i, DEPTH)
        in_copy(i, slot).wait()

        @pl.when(i >= DEPTH)
        def _wait_prev_out():
            out_copy(i - DEPTH, slot).wait()

        out_buf[slot] = in_buf[slot] + bias_ref[...]
        out_copy(i, slot).start()

        @pl.when(i + DEPTH < n_blocks)
        def _start_next_in():
            in_copy(i + DEPTH, slot).start()

        return carry

    lax.fori_loop(0, n_blocks, step, 0)
    for d in range(DEPTH):
        i_last = n_blocks - DEPTH + d
        out_copy(i_last, lax.rem(i_last, DEPTH)).wait()


def kernel(X, table):
    B = X.shape[0]
    full2d = pl.pallas_call(
        _bias_kernel,
        out_shape=jax.ShapeDtypeStruct((F_OUT, DIM), table.dtype),
    )(table)
    bias_row = full2d.reshape(1, W)
    X2 = X.reshape(B, W)
    out = pl.pallas_call(
        _stream_kernel,
        in_specs=[
            pl.BlockSpec(memory_space=pl.ANY),
            pl.BlockSpec(memory_space=pltpu.MemorySpace.VMEM),
        ],
        out_specs=pl.BlockSpec(memory_space=pl.ANY),
        out_shape=jax.ShapeDtypeStruct((B, W), X.dtype),
        scratch_shapes=[
            pltpu.VMEM((DEPTH, B_BLK, W), X.dtype),
            pltpu.VMEM((DEPTH, B_BLK, W), X.dtype),
            pltpu.SemaphoreType.DMA((DEPTH,)),
            pltpu.SemaphoreType.DMA((DEPTH,)),
        ],
        compiler_params=pltpu.CompilerParams(
            vmem_limit_bytes=100 * 1024 * 1024,
        ),
    )(X2, bias_row)
    return out.reshape(B, F_OUT, DIM)
